# Initial kernel scaffold; baseline (speedup 1.0000x reference)
import jax
import jax.numpy as jnp
from jax.experimental import pallas as pl


def kernel(edge_attr, batch, W, b, ln_w, ln_b):
    raise NotImplementedError("write your pallas kernel here")


# speedup vs baseline: 2.3326x; 2.3326x over previous
import functools

import jax
import jax.numpy as jnp
from jax import lax
from jax.experimental import pallas as pl
from jax.experimental.pallas import tpu as pltpu
from jax.experimental.pallas import tpu_sc as plsc

E = 320000
D_IN = 128
D_OUT = 256
G = 512
EPS = 1e-5

NC = 2
NS = 16
NW = NC * NS
NCHUNK = 16
EPC = E // NCHUNK
B = 400
NH = D_OUT // 128
HL = 128
NV = HL // 16


def _mm_body(x_ref, w_ref, b_ref, h_ref):
    h = jnp.dot(x_ref[...], w_ref[...], preferred_element_type=jnp.float32)
    h = jnp.maximum(h + b_ref[...], 0.0)
    h_ref[0] = h[:, :HL]
    h_ref[1] = h[:, HL:]


def _matmul_relu(edge_attr, W, b):
    BM = 3200
    return pl.pallas_call(
        _mm_body,
        grid=(E // BM,),
        in_specs=[
            pl.BlockSpec((BM, D_IN), lambda i: (i, 0)),
            pl.BlockSpec((D_IN, D_OUT), lambda i: (0, 0)),
            pl.BlockSpec((1, D_OUT), lambda i: (0, 0)),
        ],
        out_specs=pl.BlockSpec((NH, BM, HL), lambda i: (0, i, 0)),
        out_shape=jax.ShapeDtypeStruct((NH, E, HL), jnp.float32),
    )(edge_attr, W, b.reshape(1, D_OUT))


def _seg_body(h_hbm, batch_hbm, psum_hbm, pcnt_hbm, acc, cntacc, hstage,
              bstage, sem):
    wid = lax.axis_index("s") * NC + lax.axis_index("c")
    chunk = wid // NH
    fh = wid % NH
    base = chunk * EPC

    @pl.loop(0, G * HL, step=16)
    def _(g):
        acc[pl.ds(g, 16)] = jnp.zeros((16,), jnp.float32)

    @pl.loop(0, (G + 1) * 16, step=16)
    def _(g):
        cntacc[pl.ds(g, 16)] = jnp.zeros((16,), jnp.float32)

    def group_body(t, carry):
        segvec = bstage[pl.ds(16 * t, 16)]
        prev, cntvec = carry[0], carry[1]
        a = list(carry[2:])
        for k in range(16):
            e = 16 * t + k
            seg = segvec[k]
            changed = seg != prev
            av = tuple(a)
            cv = cntvec

            @pl.when(changed)
            def _(prev=prev, av=av, cv=cv):
                for j in range(NV):
                    acc[pl.ds(prev * HL + 16 * j, 16)] = av[j]
                cntacc[pl.ds(prev * 16, 16)] = cv

            keep = jnp.where(changed, 0.0, 1.0)
            for j in range(NV):
                a[j] = a[j] * keep + hstage[pl.ds(e * HL + 16 * j, 16)]
            cntvec = cntvec * keep + 1.0
            prev = seg
        return (prev, cntvec) + tuple(a)

    def chunk_body(t, carry):
        start = base + t * B
        pltpu.sync_copy(batch_hbm.at[pl.ds(start, B)], bstage)
        pltpu.sync_copy(h_hbm.at[fh, pl.ds(start * HL, B * HL)], hstage)
        return lax.fori_loop(0, B // 16, group_body, carry)

    init = (jnp.int32(G), jnp.zeros((16,), jnp.float32)) + tuple(
        jnp.zeros((16,), jnp.float32) for _ in range(NV))
    final = lax.fori_loop(0, EPC // B, chunk_body, init)

    prev, cntvec = final[0], final[1]
    for j in range(NV):
        acc[pl.ds(prev * HL + 16 * j, 16)] = final[2 + j]
    cntacc[pl.ds(prev * 16, 16)] = cntvec

    pltpu.sync_copy(acc.at[pl.ds(0, G * HL)], psum_hbm.at[wid])

    @pl.when(fh == 0)
    def _():
        pltpu.sync_copy(cntacc.at[pl.ds(0, G * 16)], pcnt_hbm.at[chunk])


def _segsum(h, batch):
    mesh = plsc.VectorSubcoreMesh(core_axis_name="c", subcore_axis_name="s")
    f = pl.kernel(
        _seg_body,
        out_type=(
            jax.ShapeDtypeStruct((NW, G * HL), jnp.float32),
            jax.ShapeDtypeStruct((NCHUNK, G * 16), jnp.float32),
        ),
        mesh=mesh,
        scratch_types=[
            pltpu.VMEM(((G + 1) * HL,), jnp.float32),
            pltpu.VMEM(((G + 1) * 16,), jnp.float32),
            pltpu.VMEM((B * HL,), jnp.float32),
            pltpu.VMEM((B,), jnp.int32),
            pltpu.SemaphoreType.DMA,
        ],
    )
    psum, pcnt = f(h.reshape(NH, E * HL), batch)
    return psum.reshape(NW, G, HL), pcnt.reshape(NCHUNK, G, 16)


def _ln_body(ps_ref, pc_ref, lnw_ref, lnb_ref, o_ref):
    s0 = ps_ref[0]
    s1 = ps_ref[1]
    for c in range(1, NCHUNK):
        s0 = s0 + ps_ref[NH * c]
        s1 = s1 + ps_ref[NH * c + 1]
    cnt = jnp.sum(pc_ref[...], axis=(0, 2)) * (1.0 / 16.0)
    mean_g = jnp.concatenate([s0, s1], axis=1) / jnp.clip(cnt, 1.0)[:, None]
    mu = jnp.mean(mean_g, axis=-1, keepdims=True)
    var = jnp.mean((mean_g - mu) ** 2, axis=-1, keepdims=True)
    o_ref[...] = ((mean_g - mu) * lax.rsqrt(var + EPS) * lnw_ref[...]
                  + lnb_ref[...])


def _layernorm(psum, pcnt, ln_w, ln_b):
    return pl.pallas_call(
        _ln_body,
        in_specs=[
            pl.BlockSpec((NW, G, HL), lambda: (0, 0, 0)),
            pl.BlockSpec((NCHUNK, G, 16), lambda: (0, 0, 0)),
            pl.BlockSpec((1, D_OUT), lambda: (0, 0)),
            pl.BlockSpec((1, D_OUT), lambda: (0, 0)),
        ],
        out_specs=pl.BlockSpec((G, D_OUT), lambda: (0, 0)),
        out_shape=jax.ShapeDtypeStruct((G, D_OUT), jnp.float32),
    )(psum, pcnt, ln_w.reshape(1, D_OUT), ln_b.reshape(1, D_OUT))


def kernel(edge_attr, batch, W, b, ln_w, ln_b):
    h = _matmul_relu(edge_attr, W, b)
    psum, pcnt = _segsum(h, batch.astype(jnp.int32))
    return _layernorm(psum, pcnt, ln_w, ln_b)


# speedup vs baseline: 4.4881x; 1.9241x over previous
import functools

import jax
import jax.numpy as jnp
from jax import lax
from jax.experimental import pallas as pl
from jax.experimental.pallas import tpu as pltpu
from jax.experimental.pallas import tpu_sc as plsc

E = 320000
D_IN = 128
D_OUT = 256
G = 512
EPS = 1e-5

NC = 2
NS = 16
NW = NC * NS
NCHUNK = 16
EPC = E // NCHUNK
B = 160
NH = D_OUT // 128
HL = 128
NV = HL // 16


def _mm_body(x_ref, w_ref, b_ref, h_ref):
    h = jnp.dot(x_ref[...], w_ref[...], preferred_element_type=jnp.float32)
    h = jnp.maximum(h + b_ref[...], 0.0)
    h_ref[0] = h[:, :HL]
    h_ref[1] = h[:, HL:]


def _matmul_relu(edge_attr, W, b):
    BM = 3200
    return pl.pallas_call(
        _mm_body,
        grid=(E // BM,),
        in_specs=[
            pl.BlockSpec((BM, D_IN), lambda i: (i, 0)),
            pl.BlockSpec((D_IN, D_OUT), lambda i: (0, 0)),
            pl.BlockSpec((1, D_OUT), lambda i: (0, 0)),
        ],
        out_specs=pl.BlockSpec((NH, BM, HL), lambda i: (0, i, 0)),
        out_shape=jax.ShapeDtypeStruct((NH, E, HL), jnp.float32),
    )(edge_attr, W, b.reshape(1, D_OUT))


def _seg_body(h_hbm, batch_hbm, psum_hbm, pcnt_hbm, acc, cntacc,
              hstage0, hstage1, bstage0, bstage1,
              hsem0, hsem1, bsem0, bsem1):
    wid = lax.axis_index("s") * NC + lax.axis_index("c")
    chunk = wid // NH
    fh = wid % NH
    base = chunk * EPC
    hbase = fh * (E * HL) + base * HL
    bufs = ((hstage0, bstage0, hsem0, bsem0),
            (hstage1, bstage1, hsem1, bsem1))

    def issue(t, buf):
        hstage, bstage, hsem, bsem = bufs[buf]
        pltpu.make_async_copy(
            batch_hbm.at[pl.ds(base + t * B, B)], bstage, bsem).start()
        pltpu.make_async_copy(
            h_hbm.at[pl.ds(hbase + t * B * HL, B * HL)], hstage, hsem).start()

    def wait(buf):
        hstage, bstage, hsem, bsem = bufs[buf]
        pltpu.make_async_copy(
            batch_hbm.at[pl.ds(base, B)], bstage, bsem).wait()
        pltpu.make_async_copy(
            h_hbm.at[pl.ds(hbase, B * HL)], hstage, hsem).wait()

    @pl.loop(0, G * HL, step=16)
    def _(g):
        acc[pl.ds(g, 16)] = jnp.zeros((16,), jnp.float32)

    @pl.loop(0, (G + 1) * 16, step=16)
    def _(g):
        cntacc[pl.ds(g, 16)] = jnp.zeros((16,), jnp.float32)

    def make_group_body(hstage, bstage):
        def group_body(t, carry):
            segvec = bstage[pl.ds(16 * t, 16)]
            prev, cntvec = carry[0], carry[1]
            a = list(carry[2:])
            for k in range(16):
                e = 16 * t + k
                seg = segvec[k]
                changed = seg != prev
                av = tuple(a)
                cv = cntvec

                @pl.when(changed)
                def _(prev=prev, av=av, cv=cv):
                    for j in range(NV):
                        acc[pl.ds(prev * HL + 16 * j, 16)] = av[j]
                    cntacc[pl.ds(prev * 16, 16)] = cv

                keep = jnp.where(changed, 0.0, 1.0)
                for j in range(NV):
                    a[j] = a[j] * keep + hstage[pl.ds(e * HL + 16 * j, 16)]
                cntvec = cntvec * keep + 1.0
                prev = seg
            return (prev, cntvec) + tuple(a)
        return group_body

    NT = EPC // B

    def pair_body(tt, carry):
        t0 = 2 * tt

        wait(0)
        carry = lax.fori_loop(0, B // 16,
                              make_group_body(hstage0, bstage0), carry)

        @pl.when(t0 + 2 < NT)
        def _():
            issue(t0 + 2, 0)

        wait(1)
        carry = lax.fori_loop(0, B // 16,
                              make_group_body(hstage1, bstage1), carry)

        @pl.when(t0 + 3 < NT)
        def _():
            issue(t0 + 3, 1)

        return carry

    issue(0, 0)
    issue(1, 1)
    init = (jnp.int32(G), jnp.zeros((16,), jnp.float32)) + tuple(
        jnp.zeros((16,), jnp.float32) for _ in range(NV))
    final = lax.fori_loop(0, NT // 2, pair_body, init)
    if NT % 2:
        wait(0)
        final = lax.fori_loop(0, B // 16,
                              make_group_body(hstage0, bstage0), final)

    prev, cntvec = final[0], final[1]
    for j in range(NV):
        acc[pl.ds(prev * HL + 16 * j, 16)] = final[2 + j]
    cntacc[pl.ds(prev * 16, 16)] = cntvec

    pltpu.sync_copy(acc.at[pl.ds(0, G * HL)],
                    psum_hbm.at[pl.ds(wid * G * HL, G * HL)])

    @pl.when(fh == 0)
    def _():
        pltpu.sync_copy(cntacc.at[pl.ds(0, G * 16)], pcnt_hbm.at[chunk])


def _segsum(h_flat, batch):
    mesh = plsc.VectorSubcoreMesh(core_axis_name="c", subcore_axis_name="s")
    f = pl.kernel(
        _seg_body,
        out_type=(
            jax.ShapeDtypeStruct((NW * G * HL,), jnp.float32),
            jax.ShapeDtypeStruct((NCHUNK, G * 16), jnp.float32),
        ),
        mesh=mesh,
        scratch_types=[
            pltpu.VMEM(((G + 1) * HL,), jnp.float32),
            pltpu.VMEM(((G + 1) * 16,), jnp.float32),
            pltpu.VMEM((B * HL,), jnp.float32),
            pltpu.VMEM((B * HL,), jnp.float32),
            pltpu.VMEM((B,), jnp.int32),
            pltpu.VMEM((B,), jnp.int32),
            pltpu.SemaphoreType.DMA,
            pltpu.SemaphoreType.DMA,
            pltpu.SemaphoreType.DMA,
            pltpu.SemaphoreType.DMA,
        ],
    )
    psum, pcnt = f(h_flat, batch)
    return psum.reshape(NW, G, HL), pcnt.reshape(NCHUNK, G, 16)


def _ln_body(ps_ref, pc_ref, lnw_ref, lnb_ref, o_ref):
    s0 = ps_ref[0]
    s1 = ps_ref[1]
    for c in range(1, NCHUNK):
        s0 = s0 + ps_ref[NH * c]
        s1 = s1 + ps_ref[NH * c + 1]
    cnt = jnp.sum(pc_ref[...], axis=(0, 2)) * (1.0 / 16.0)
    mean_g = jnp.concatenate([s0, s1], axis=1) / jnp.clip(cnt, 1.0)[:, None]
    mu = jnp.mean(mean_g, axis=-1, keepdims=True)
    var = jnp.mean((mean_g - mu) ** 2, axis=-1, keepdims=True)
    o_ref[...] = ((mean_g - mu) * lax.rsqrt(var + EPS) * lnw_ref[...]
                  + lnb_ref[...])


def _layernorm(psum, pcnt, ln_w, ln_b):
    return pl.pallas_call(
        _ln_body,
        in_specs=[
            pl.BlockSpec((NW, G, HL), lambda: (0, 0, 0)),
            pl.BlockSpec((NCHUNK, G, 16), lambda: (0, 0, 0)),
            pl.BlockSpec((1, D_OUT), lambda: (0, 0)),
            pl.BlockSpec((1, D_OUT), lambda: (0, 0)),
        ],
        out_specs=pl.BlockSpec((G, D_OUT), lambda: (0, 0)),
        out_shape=jax.ShapeDtypeStruct((G, D_OUT), jnp.float32),
    )(psum, pcnt, ln_w.reshape(1, D_OUT), ln_b.reshape(1, D_OUT))


def kernel(edge_attr, batch, W, b, ln_w, ln_b):
    h = _matmul_relu(edge_attr, W, b)
    psum, pcnt = _segsum(h.reshape(NH * E * HL), batch.astype(jnp.int32))
    return _layernorm(psum, pcnt, ln_w, ln_b)
